# Initial kernel scaffold; baseline (speedup 1.0000x reference)
import jax
import jax.numpy as jnp
from jax.experimental import pallas as pl


def kernel(x, shared, W_mlp1, b_mlp1, W_mlp2, b_mlp2, Wg, Wf, P0, P1, P2, Wout):
    raise NotImplementedError("write your pallas kernel here")


# speedup vs baseline: 2.8171x; 2.8171x over previous
import functools

import jax
import jax.numpy as jnp
from jax.experimental import pallas as pl
from jax.experimental.pallas import tpu as pltpu

DIM = 64
RANK = 16
E = 8
K = 2
H = 512
W = 512

ROWS_A = 32
ROWS_C = 32
NA = H // ROWS_A
NC = H // ROWS_C
PC = ROWS_C * W

_PREC = jax.lax.Precision.HIGHEST


def _gelu_exact(v):
    return 0.5 * v * (1.0 + jax.lax.erf(v * 0.7071067811865476))


def _stats_kernel(x_ref, sums_ref, t_carry, x_carry):
    i = pl.program_id(0)
    xc = x_ref[0]
    T = ROWS_A

    col = jax.lax.broadcasted_iota(jnp.int32, (DIM, T, W), 2)
    left = jnp.where(col == 0, 0.0, jnp.roll(xc, 1, axis=2))
    right = jnp.where(col == W - 1, 0.0, jnp.roll(xc, -1, axis=2))
    t = xc + left + right

    @pl.when(i == 0)
    def _init():
        t_carry[...] = jnp.zeros_like(t_carry)
        x_carry[...] = jnp.zeros_like(x_carry)
        sums_ref[...] = jnp.zeros_like(sums_ref)

    ext_t = jnp.concatenate([t_carry[...], t], axis=1)
    box = ext_t[:, 0:T] + ext_t[:, 1:T + 1] + ext_t[:, 2:T + 2]
    ext_x = jnp.concatenate([x_carry[...], xc[:, :T - 1]], axis=1)
    hp = 9.0 * ext_x - box
    g = _gelu_exact(hp)
    row = jax.lax.broadcasted_iota(jnp.int32, (DIM, T, W), 1)
    g = jnp.where((i == 0) & (row == 0), 0.0, g)
    gsum = jnp.sum(g, axis=(1, 2))
    xsum = jnp.sum(xc, axis=(1, 2))

    @pl.when(i == NA - 1)
    def _last_row():
        box_l = t[:, T - 2] + t[:, T - 1]
        hp_l = 9.0 * xc[:, T - 1] - box_l
        sums_ref[0, :] += jnp.sum(_gelu_exact(hp_l), axis=1)

    sums_ref[0, :] += gsum
    sums_ref[1, :] += xsum

    t_carry[...] = t[:, T - 2:T]
    x_carry[...] = xc[:, T - 1:T]


def _apply_kernel(x_ref, s_ref, a_ref, b_ref, m_ref, wx_ref, o_ref):
    X = x_ref[...]
    S = s_ref[...]
    a = jnp.dot(a_ref[...], X, preferred_element_type=jnp.float32,
                precision=_PREC)
    b = jnp.dot(b_ref[...], S, preferred_element_type=jnp.float32,
                precision=_PREC)
    y = a * (b * jax.nn.sigmoid(b))
    o_ref[...] = (
        jnp.dot(m_ref[...], y, preferred_element_type=jnp.float32,
                precision=_PREC)
        + jnp.dot(wx_ref[...], X, preferred_element_type=jnp.float32,
                  precision=_PREC))


@functools.partial(jax.jit, static_argnums=())
def kernel(x, shared, W_mlp1, b_mlp1, W_mlp2, b_mlp2, Wg, Wf, P0, P1, P2, Wout):
    sums = pl.pallas_call(
        _stats_kernel,
        grid=(NA,),
        in_specs=[pl.BlockSpec((1, DIM, ROWS_A, W), lambda i: (0, 0, i, 0))],
        out_specs=pl.BlockSpec((2, DIM), lambda i: (0, 0)),
        out_shape=jax.ShapeDtypeStruct((2, DIM), jnp.float32),
        scratch_shapes=[
            pltpu.VMEM((DIM, 2, W), jnp.float32),
            pltpu.VMEM((DIM, 1, W), jnp.float32),
        ],
    )(x)
    inv_hw = 1.0 / (H * W)
    f0 = sums[0] * inv_hw
    pooled = sums[1] * inv_hw

    f = _gelu_exact(f0 @ W_mlp1.T + b_mlp1) @ W_mlp2.T + b_mlp2
    logits = pooled @ Wg.T + f @ Wf.T
    scores = jax.nn.softmax(logits)
    topv, topi = jax.lax.top_k(scores, K)
    v0, v1 = topv[0], topv[1]
    e0, e1 = topi[0], topi[1]

    A = jnp.concatenate([P0[e0], P0[e1]], axis=0)
    Bm = jnp.concatenate([P1[e0], P1[e1]], axis=0)
    C2 = jnp.concatenate([v0 * P2[e0], v1 * P2[e1]], axis=1)
    M = Wout @ C2
    Wx = (v0 + v1) * Wout

    x2 = x.reshape(DIM, H * W)
    s2 = shared.reshape(DIM, H * W)
    out2 = pl.pallas_call(
        _apply_kernel,
        grid=(NC,),
        in_specs=[
            pl.BlockSpec((DIM, PC), lambda i: (0, i)),
            pl.BlockSpec((DIM, PC), lambda i: (0, i)),
            pl.BlockSpec((2 * RANK, DIM), lambda i: (0, 0)),
            pl.BlockSpec((2 * RANK, DIM), lambda i: (0, 0)),
            pl.BlockSpec((DIM, 2 * RANK), lambda i: (0, 0)),
            pl.BlockSpec((DIM, DIM), lambda i: (0, 0)),
        ],
        out_specs=pl.BlockSpec((DIM, PC), lambda i: (0, i)),
        out_shape=jax.ShapeDtypeStruct((DIM, H * W), jnp.float32),
    )(x2, s2, A, Bm, M, Wx)
    return out2.reshape(1, DIM, H, W)
